# baseline (speedup 1.0000x reference)
import functools

import jax
import jax.numpy as jnp
from jax import lax
from jax.experimental import pallas as pl
from jax.experimental.pallas import tpu as pltpu
from jax.experimental.pallas import tpu_sc as plsc

_N = 10000
_E = 160000
_B = 4
_NC = 2
_EC = _E // _NC
_BE = 3200
_NBEC = _EC // _BE
_BN = 2000
_NBN = _N // _BN
_GW = 128
_ZCH = 80
_NZCH = _N // _ZCH

_BINS = (1.0, 2.0, 3.0, 4.0, 5.0, 10.0, 15.0)


def _full_spec(shape):
    return pl.BlockSpec(shape, lambda *_: (0,) * len(shape))


def _tcol(row):
    return lax.dot_general(row, jnp.ones((1, 1), jnp.float32),
                           (((0,), (0,)), ((), ())))


def _iota8f(n):
    return lax.broadcasted_iota(jnp.int32, (n, 8), 1).astype(jnp.float32)


def _enc_node_body(xin, w1, b1, w2, b2, out):
    h = jnp.maximum(xin[...] @ w1[...] + b1[...], 0.0)
    out[...] = jnp.maximum(h @ w2[...] + b2[...], 0.0)


def _enc_node(xin, w1, b1, w2, b2):
    return pl.pallas_call(
        _enc_node_body,
        grid=(_NBN,),
        in_specs=[
            pl.BlockSpec((_BN, 24), lambda i: (i, 0)),
            _full_spec(w1.shape), _full_spec(b1.shape),
            _full_spec(w2.shape), _full_spec(b2.shape),
        ],
        out_specs=pl.BlockSpec((_BN, 128), lambda i: (i, 0)),
        out_shape=jax.ShapeDtypeStruct((_N, 128), jnp.float32),
    )(xin, w1, b1, w2, b2)


def _enc_edge_body(eaT, dist, srcr, dstr, w1, b1, w2, b2, table, out):
    h = jnp.maximum(
        lax.dot_general(eaT[...], w1[...], (((0,), (0,)), ((), ())))
        + b1[...], 0.0)
    em = jnp.maximum(h @ w2[...] + b2[...], 0.0)
    sep_row = jnp.abs(srcr[...] - dstr[...])
    code_row = jnp.zeros_like(sep_row)
    for b in _BINS:
        code_row = code_row + (sep_row > b).astype(jnp.float32)
    code = _tcol(code_row)
    oh = (code == _iota8f(_BE)).astype(jnp.float32)
    sep_emb = oh @ table[...]
    centers = lax.broadcasted_iota(jnp.int32, (1, 16), 1).astype(
        jnp.float32) * (20.0 / 15.0)
    r = _tcol(dist[...]) - centers
    rbf = jnp.exp(-(r * r))
    out[...] = jnp.concatenate([em, sep_emb, rbf], axis=1)


def _enc_edge(eaT, dist2, src2f, dst2f, w1, b1, w2, b2, table, co):
    return pl.pallas_call(
        _enc_edge_body,
        grid=(_NBEC,),
        in_specs=[
            pl.BlockSpec((4, _BE), lambda i: (0, co + i)),
            pl.BlockSpec((1, _BE), lambda i: (0, co + i)),
            pl.BlockSpec((1, _BE), lambda i: (0, co + i)),
            pl.BlockSpec((1, _BE), lambda i: (0, co + i)),
            _full_spec(w1.shape), _full_spec(b1.shape),
            _full_spec(w2.shape), _full_spec(b2.shape),
            _full_spec(table.shape),
        ],
        out_specs=pl.BlockSpec((_BE, 144), lambda i: (i, 0)),
        out_shape=jax.ShapeDtypeStruct((_EC, 144), jnp.float32),
    )(eaT, dist2, src2f, dst2f, w1, b1, w2, b2, table)


def _prep0_body(x, wex, be, out):
    out[...] = x[...] @ wex[...] + be[...]


def _prep0(x, wex, be):
    return pl.pallas_call(
        _prep0_body,
        grid=(_NBN,),
        in_specs=[
            pl.BlockSpec((_BN, 128), lambda i: (i, 0)),
            _full_spec(wex.shape), _full_spec(be.shape),
        ],
        out_specs=pl.BlockSpec((_BN, 128), lambda i: (i, 0)),
        out_shape=jax.ShapeDtypeStruct((_N, 128), jnp.float32),
    )(x, wex, be)


def _glob_update(u_ref, acce, accn, wge, wgx, wgu, bg):
    eg = acce[0:4, :] / jnp.maximum(acce[8:12, :], 1.0)
    xg = accn[0:4, :] / jnp.maximum(accn[8:12, :], 1.0)
    uu = u_ref[0:4, 0:32]
    un = jnp.maximum(eg @ wge[...] + xg @ wgx[...] + uu @ wgu[...] + bg[...],
                     0.0)
    return jnp.concatenate([un, jnp.zeros((4, 32), jnp.float32)], axis=0)


def _prep_body(x, batchr, u, acca, accb, accn, wge, wgx, wgu, bg, wex, weu,
               be, t_out, u_out):
    i = pl.program_id(0)
    un8 = _glob_update(u, acca[...] + accb[...], accn[...],
                       wge, wgx, wgu, bg)
    term = un8 @ weu[...]
    bcol = batchr[0].astype(jnp.float32)
    oh = (bcol == _iota8f(_BN)).astype(jnp.float32)
    t_out[...] = x[...] @ wex[...] + oh @ term + be[...]

    @pl.when(i == 0)
    def _():
        u_out[...] = jnp.concatenate(
            [un8, jnp.zeros((8, 96), jnp.float32)], axis=1)


def _prep(x, batch_r, u, acca, accb, accn, wge, wgx, wgu, bg, wex, weu, be):
    return pl.pallas_call(
        _prep_body,
        grid=(_NBN,),
        in_specs=[
            pl.BlockSpec((_BN, 128), lambda i: (i, 0)),
            pl.BlockSpec((1, _BN, 1), lambda i: (i, 0, 0)),
            _full_spec((8, 128)), _full_spec((16, 128)),
            _full_spec((16, 128)), _full_spec((16, 128)),
            _full_spec(wge.shape), _full_spec(wgx.shape),
            _full_spec(wgu.shape), _full_spec(bg.shape),
            _full_spec(wex.shape), _full_spec(weu.shape), _full_spec(be.shape),
        ],
        out_specs=[
            pl.BlockSpec((_BN, 128), lambda i: (i, 0)),
            pl.BlockSpec((8, 128), lambda i: (0, 0)),
        ],
        out_shape=[
            jax.ShapeDtypeStruct((_N, 128), jnp.float32),
            jax.ShapeDtypeStruct((8, 128), jnp.float32),
        ],
    )(x, batch_r, u, acca, accb, accn, wge, wgx, wgu, bg, wex, weu, be)


def _edge_body(e, g, srcr, lo, wee, e_out, acc_out):
    i = pl.program_id(0)
    en = jnp.maximum(g[...] + e[...] @ wee[...], 0.0)
    e_out[...] = en
    code_row = jnp.zeros((1, _BE), jnp.float32)
    for gidx in range(1, _B):
        code_row = code_row + (srcr[...] >= lo[0:1, gidx:gidx + 1]).astype(
            jnp.float32)
    oh = (_tcol(code_row) == _iota8f(_BE)).astype(jnp.float32)
    part = lax.dot_general(oh, en, (((0,), (0,)), ((), ())))
    cnt = lax.dot_general(oh, jnp.ones((_BE, 128), jnp.float32),
                          (((0,), (0,)), ((), ())))
    blk = jnp.concatenate([part, cnt], axis=0)

    @pl.when(i == 0)
    def _():
        acc_out[...] = blk

    @pl.when(i > 0)
    def _():
        acc_out[...] = acc_out[...] + blk


def _edge(e, g, src2f, lo, wee, co):
    ie = e.shape[1]
    return pl.pallas_call(
        _edge_body,
        grid=(_NBEC,),
        in_specs=[
            pl.BlockSpec((_BE, ie), lambda i: (i, 0)),
            pl.BlockSpec((_BE, 128), lambda i: (i, 0)),
            pl.BlockSpec((1, _BE), lambda i: (0, co + i)),
            _full_spec((1, 8)),
            _full_spec(wee.shape),
        ],
        out_specs=[
            pl.BlockSpec((_BE, 128), lambda i: (i, 0)),
            pl.BlockSpec((16, 128), lambda i: (0, 0)),
        ],
        out_shape=[
            jax.ShapeDtypeStruct((_EC, 128), jnp.float32),
            jax.ShapeDtypeStruct((16, 128), jnp.float32),
        ],
    )(e, g, src2f, lo, wee)


def _node_body(x, spa, spb, dp, batchr, u, wnx, wnm, wnu, bn, wr, br,
               x_out, acc_out, last):
    i = pl.program_id(0)
    deg = jnp.maximum(dp[0, :, 0:1] + dp[1, :, 0:1], 1.0)
    msg = (spa[0] + spa[1] + spb[0] + spb[1]) / deg
    uterm = u[0:8, 0:32] @ wnu[...]
    bcol = batchr[0].astype(jnp.float32)
    oh = (bcol == _iota8f(_BN)).astype(jnp.float32)
    xn = jnp.maximum(
        x[...] @ wnx[...] + msg @ wnm[...] + oh @ uterm + bn[...], 0.0)
    if last:
        x_out[...] = jax.nn.sigmoid(xn @ wr[...] + br[...])
    else:
        x_out[...] = xn
    part = lax.dot_general(oh, xn, (((0,), (0,)), ((), ())))
    cnt = lax.dot_general(oh, jnp.ones((_BN, 128), jnp.float32),
                          (((0,), (0,)), ((), ())))
    blk = jnp.concatenate([part, cnt], axis=0)

    @pl.when(i == 0)
    def _():
        acc_out[...] = blk

    @pl.when(i > 0)
    def _():
        acc_out[...] = acc_out[...] + blk


def _node(x, spa, spb, dp, batch_r, u, wnx, wnm, wnu, bn, wr, br, last):
    return pl.pallas_call(
        functools.partial(_node_body, last=last),
        grid=(_NBN,),
        in_specs=[
            pl.BlockSpec((_BN, 128), lambda i: (i, 0)),
            pl.BlockSpec((2, _BN, 128), lambda i: (0, i, 0)),
            pl.BlockSpec((2, _BN, 128), lambda i: (0, i, 0)),
            pl.BlockSpec((2, _BN, 128), lambda i: (0, i, 0)),
            pl.BlockSpec((1, _BN, 1), lambda i: (i, 0, 0)),
            _full_spec((8, 128)),
            _full_spec(wnx.shape), _full_spec(wnm.shape),
            _full_spec(wnu.shape), _full_spec(bn.shape),
            _full_spec(wr.shape), _full_spec(br.shape),
        ],
        out_specs=[
            pl.BlockSpec((_BN, 128), lambda i: (i, 0)),
            pl.BlockSpec((16, 128), lambda i: (0, 0)),
        ],
        out_shape=[
            jax.ShapeDtypeStruct((_N, 128), jnp.float32),
            jax.ShapeDtypeStruct((16, 128), jnp.float32),
        ],
    )(x, spa, spb, dp, batch_r, u, wnx, wnm, wnu, bn, wr, br)


def _final_body(u, acca, accb, accn, wge, wgx, wgu, bg, wu, bu, out):
    un8 = _glob_update(u, acca[...] + accb[...], accn[...],
                       wge, wgx, wgu, bg)
    y = jax.nn.sigmoid(un8[0:4, :] @ wu[...] + bu[...])
    out[...] = jnp.concatenate([y, jnp.zeros((4, 128), jnp.float32)], axis=0)


def _final(u, acca, accb, accn, wge, wgx, wgu, bg, wu, bu):
    return pl.pallas_call(
        _final_body,
        grid=(1,),
        in_specs=[
            _full_spec((8, 128)), _full_spec((16, 128)),
            _full_spec((16, 128)), _full_spec((16, 128)),
            _full_spec(wge.shape), _full_spec(wgx.shape),
            _full_spec(wgu.shape), _full_spec(bg.shape),
            _full_spec(wu.shape), _full_spec(bu.shape),
        ],
        out_specs=pl.BlockSpec((8, 128), lambda i: (0, 0)),
        out_shape=jax.ShapeDtypeStruct((8, 128), jnp.float32),
    )(u, acca, accb, accn, wge, wgx, wgu, bg, wu, bu)


_MESH = dict(core_axis_name="core", subcore_axis_name="subcore")


def _sc_gather(table, idx2, co):
    mesh = plsc.VectorSubcoreMesh(**_MESH)

    @functools.partial(
        pl.kernel,
        out_type=jax.ShapeDtypeStruct((_EC, 128), jnp.float32),
        mesh=mesh)
    def k(x_hbm, i_hbm, o_hbm):
        def body(i_vmem, o_vmem):
            pltpu.sync_copy(x_hbm.at[i_vmem.at[0]], o_vmem)

        pltpu.emit_pipeline(
            body,
            grid=(_EC // _GW,),
            in_specs=[pl.BlockSpec((1, _GW), lambda i: (0, co + i))],
            out_specs=[pl.BlockSpec((_GW, 128), lambda i: (i, 0))],
            core_axis_name=("core", "subcore"),
            dimension_semantics=(pltpu.PARALLEL,),
        )(i_hbm, o_hbm)

    return k(table, idx2)


def _sc_scatter_add(vals, idx2, zrows, co):
    mesh = plsc.VectorSubcoreMesh(**_MESH)

    @functools.partial(
        pl.kernel,
        out_type=jax.ShapeDtypeStruct((2, _N, 128), jnp.float32),
        mesh=mesh,
        scratch_types=[pltpu.VMEM_SHARED((_N, 128), jnp.float32)])
    def k(v_hbm, i_hbm, z_hbm, o_hbm, acc):
        c = lax.axis_index("core")
        s = lax.axis_index("subcore")

        @pl.loop(s, _NZCH, step=16)
        def _(i):
            pltpu.sync_copy(z_hbm, acc.at[pl.ds(i * _ZCH, _ZCH)])

        plsc.subcore_barrier()

        def body(v_vmem, i_vmem):
            pltpu.sync_copy(v_vmem, acc.at[i_vmem.at[0]], add=True)

        pltpu.emit_pipeline(
            body,
            grid=(_EC // _GW,),
            in_specs=[
                pl.BlockSpec((_GW, 128), lambda i: (i, 0)),
                pl.BlockSpec((1, _GW), lambda i: (0, co + i)),
            ],
            out_specs=[],
            core_axis_name=("core", "subcore"),
            dimension_semantics=(pltpu.PARALLEL,),
        )(v_hbm, i_hbm)
        plsc.subcore_barrier()

        @pl.loop(s, _NZCH, step=16)
        def _(i):
            pltpu.sync_copy(acc.at[pl.ds(i * _ZCH, _ZCH)],
                            o_hbm.at[c, pl.ds(i * _ZCH, _ZCH)])

    return k(vals, idx2, zrows)


def _sc_degree(idx2, zrows16):
    mesh = plsc.VectorSubcoreMesh(**_MESH)

    @functools.partial(
        pl.kernel,
        out_type=jax.ShapeDtypeStruct((2, _N, 128), jnp.float32),
        mesh=mesh,
        scratch_types=[pltpu.VMEM_SHARED((_N, 128), jnp.float32),
                       pltpu.VMEM((_GW, 128), jnp.float32)])
    def k(i_hbm, z_hbm, o_hbm, acc, ones_v):
        c = lax.axis_index("core")
        s = lax.axis_index("subcore")

        @pl.loop(0, _GW)
        def _(r):
            @pl.loop(0, 128, step=16)
            def _(j):
                ones_v[r, pl.ds(j, 16)] = jnp.ones((16,), jnp.float32)

        @pl.loop(s, _NZCH, step=16)
        def _(i):
            pltpu.sync_copy(z_hbm, acc.at[pl.ds(i * _ZCH, _ZCH)])

        plsc.subcore_barrier()

        def body(i_vmem):
            pltpu.sync_copy(ones_v, acc.at[i_vmem.at[0]], add=True)

        pltpu.emit_pipeline(
            body,
            grid=(_E // _GW,),
            in_specs=[pl.BlockSpec((1, _GW), lambda i: (0, i))],
            out_specs=[],
            core_axis_name=("core", "subcore"),
            dimension_semantics=(pltpu.PARALLEL,),
        )(i_hbm)
        plsc.subcore_barrier()

        @pl.loop(s, _NZCH, step=16)
        def _(i):
            pltpu.sync_copy(acc.at[pl.ds(i * _ZCH, _ZCH)],
                            o_hbm.at[c, pl.ds(i * _ZCH, _ZCH)])

    return k(idx2, zrows16)


def kernel(x, msa_feats, edge_attr, distances, edge_index, batch, params):
    p = params
    src = edge_index[0]
    dst = edge_index[1]

    xin = jnp.concatenate([x, msa_feats], axis=1)
    src2 = src.reshape(1, _E)
    dst2 = dst.reshape(1, _E)
    src2f = src.astype(jnp.float32).reshape(1, _E)
    dst2f = dst.astype(jnp.float32).reshape(1, _E)
    dist2 = distances.reshape(1, _E)
    eaT = edge_attr.T
    batch_r = batch.reshape(_NBN, _BN, 1)

    lo4 = jnp.searchsorted(batch, jnp.arange(_B, dtype=jnp.int32)).astype(
        jnp.float32)
    lo = jnp.concatenate([lo4, jnp.full((4,), float(_N + 1), jnp.float32)]
                         ).reshape(1, 8)

    zrows = jnp.zeros((_ZCH, 128), jnp.float32)

    def row(b):
        return b.reshape(1, -1)

    (enw1, enb1), (enw2, enb2) = p["enc_node"]
    (eew1, eeb1), (eew2, eeb2) = p["enc_edge"]

    x_h = _enc_node(xin, enw1, row(enb1), enw2, row(enb2))
    e_ch = [_enc_edge(eaT, dist2, src2f, dst2f,
                      eew1, row(eeb1), eew2, row(eeb2), p["sep_table"],
                      c * _NBEC)
            for c in range(_NC)]

    degp = _sc_degree(dst2, zrows)

    u = jnp.zeros((8, 128), jnp.float32)
    wr, br = p["ro_node"]
    wr_pad = jnp.pad(wr, ((0, 0), (0, 126)))
    br_pad = jnp.pad(row(br), ((0, 0), (0, 126)))
    zw = jnp.zeros((128, 128), jnp.float32)
    zb = jnp.zeros((1, 128), jnp.float32)

    acc_e = acc_n = None
    for li in range(6):
        (we, be), (wn, bn), (wg, bg) = p["mp"][li]
        ie = 144 if li == 0 else 128
        we_x = we[0:128]
        we_e = we[128:128 + ie]
        we_u = we[128 + ie:]

        if li == 0:
            t = _prep0(x_h, we_x, row(be))
        else:
            wg_p, bg_p = p["mp"][li - 1][2]
            t, u = _prep(x_h, batch_r, u, acc_e[0], acc_e[1], acc_n,
                         wg_p[0:128], wg_p[128:256], wg_p[256:288],
                         row(bg_p), we_x, we_u, row(be))

        g_ch = [_sc_gather(t, src2, c * (_EC // _GW)) for c in range(_NC)]
        acc_e, sp_ch = [], []
        for c in range(_NC):
            en, acc = _edge(e_ch[c], g_ch[c], src2f, lo, we_e, c * _NBEC)
            e_ch[c] = en
            acc_e.append(acc)
            sp_ch.append(_sc_scatter_add(en, dst2, zrows, c * (_EC // _GW)))
        last = li == 5
        x_h, acc_n = _node(
            x_h, sp_ch[0], sp_ch[1], degp, batch_r, u,
            wn[0:128], wn[128:256], wn[256:288], row(bn),
            wr_pad if last else zw, br_pad if last else zb, last)

    (wg_l, bg_l) = p["mp"][5][2]
    wu, bu = p["ro_glob"]
    wu_pad = jnp.pad(wu, ((0, 0), (0, 123)))
    bu_pad = jnp.pad(row(bu), ((0, 0), (0, 123)))
    yg = _final(u, acc_e[0], acc_e[1], acc_n,
                wg_l[0:128], wg_l[128:256], wg_l[256:288],
                row(bg_l), wu_pad, bu_pad)

    return x_h[:, 0:2], yg[0:4, 0:5]


# speedup vs baseline: 1.1135x; 1.1135x over previous
import functools

import jax
import jax.numpy as jnp
from jax import lax
from jax.experimental import pallas as pl
from jax.experimental.pallas import tpu as pltpu
from jax.experimental.pallas import tpu_sc as plsc

_N = 10000
_E = 160000
_B = 4
_NC = 2
_EC = _E // _NC
_BE = 3200
_NBEC = _EC // _BE
_BN = 2000
_NBN = _N // _BN
_GW = 128
_ZCH = 1000
_NZCH = _N // _ZCH

_BINS = (1.0, 2.0, 3.0, 4.0, 5.0, 10.0, 15.0)


def _full_spec(shape):
    return pl.BlockSpec(shape, lambda *_: (0,) * len(shape))


def _tcol(row):
    return lax.dot_general(row, jnp.ones((1, 1), jnp.float32),
                           (((0,), (0,)), ((), ())))


def _iota8f(n):
    return lax.broadcasted_iota(jnp.int32, (n, 8), 1).astype(jnp.float32)


def _enc_node_body(xin, w1, b1, w2, b2, wex, be, x_out, t_out):
    h = jnp.maximum(xin[...] @ w1[...] + b1[...], 0.0)
    xn = jnp.maximum(h @ w2[...] + b2[...], 0.0)
    x_out[...] = xn
    t_out[...] = xn @ wex[...] + be[...]


def _enc_node(xin, w1, b1, w2, b2, wex, be):
    return pl.pallas_call(
        _enc_node_body,
        grid=(_NBN,),
        in_specs=[
            pl.BlockSpec((_BN, 24), lambda i: (i, 0)),
            _full_spec(w1.shape), _full_spec(b1.shape),
            _full_spec(w2.shape), _full_spec(b2.shape),
            _full_spec(wex.shape), _full_spec(be.shape),
        ],
        out_specs=[
            pl.BlockSpec((_BN, 128), lambda i: (i, 0)),
            pl.BlockSpec((_BN, 128), lambda i: (i, 0)),
        ],
        out_shape=[
            jax.ShapeDtypeStruct((_N, 128), jnp.float32),
            jax.ShapeDtypeStruct((_N, 128), jnp.float32),
        ],
    )(xin, w1, b1, w2, b2, wex, be)


def _enc_edge_body(eaT, dist, srcr, dstr, w1, b1, w2, b2, table, out):
    h = jnp.maximum(
        lax.dot_general(eaT[...], w1[...], (((0,), (0,)), ((), ())))
        + b1[...], 0.0)
    em = jnp.maximum(h @ w2[...] + b2[...], 0.0)
    sep_row = jnp.abs(srcr[...] - dstr[...])
    code_row = jnp.zeros_like(sep_row)
    for b in _BINS:
        code_row = code_row + (sep_row > b).astype(jnp.float32)
    code = _tcol(code_row)
    oh = (code == _iota8f(_BE)).astype(jnp.float32)
    sep_emb = oh @ table[...]
    centers = lax.broadcasted_iota(jnp.int32, (1, 16), 1).astype(
        jnp.float32) * (20.0 / 15.0)
    r = _tcol(dist[...]) - centers
    rbf = jnp.exp(-(r * r))
    out[...] = jnp.concatenate([em, sep_emb, rbf], axis=1)


def _enc_edge(eaT, dist2, src2f, dst2f, w1, b1, w2, b2, table, co):
    return pl.pallas_call(
        _enc_edge_body,
        grid=(_NBEC,),
        in_specs=[
            pl.BlockSpec((4, _BE), lambda i: (0, co + i)),
            pl.BlockSpec((1, _BE), lambda i: (0, co + i)),
            pl.BlockSpec((1, _BE), lambda i: (0, co + i)),
            pl.BlockSpec((1, _BE), lambda i: (0, co + i)),
            _full_spec(w1.shape), _full_spec(b1.shape),
            _full_spec(w2.shape), _full_spec(b2.shape),
            _full_spec(table.shape),
        ],
        out_specs=pl.BlockSpec((_BE, 144), lambda i: (i, 0)),
        out_shape=jax.ShapeDtypeStruct((_EC, 144), jnp.float32),
    )(eaT, dist2, src2f, dst2f, w1, b1, w2, b2, table)


def _glob_update(u_ref, acce, accn, wge, wgx, wgu, bg):
    eg = acce[0:4, :] / jnp.maximum(acce[8:12, :], 1.0)
    xg = accn[0:4, :] / jnp.maximum(accn[8:12, :], 1.0)
    uu = u_ref[0:4, 0:32]
    un = jnp.maximum(eg @ wge[...] + xg @ wgx[...] + uu @ wgu[...] + bg[...],
                     0.0)
    return jnp.concatenate([un, jnp.zeros((4, 32), jnp.float32)], axis=0)


def _uker_body(u, acca, accb, accn, wge, wgx, wgu, bg, weu, u_out, ut_out):
    un8 = _glob_update(u, acca[...] + accb[...], accn[...],
                       wge, wgx, wgu, bg)
    u_out[...] = jnp.concatenate(
        [un8, jnp.zeros((8, 96), jnp.float32)], axis=1)
    ut_out[...] = un8 @ weu[...]


def _uker(u, acca, accb, accn, wge, wgx, wgu, bg, weu):
    return pl.pallas_call(
        _uker_body,
        grid=(1,),
        in_specs=[
            _full_spec((8, 128)), _full_spec((16, 128)),
            _full_spec((16, 128)), _full_spec((16, 128)),
            _full_spec(wge.shape), _full_spec(wgx.shape),
            _full_spec(wgu.shape), _full_spec(bg.shape),
            _full_spec(weu.shape),
        ],
        out_specs=[
            pl.BlockSpec((8, 128), lambda i: (0, 0)),
            pl.BlockSpec((8, 128), lambda i: (0, 0)),
        ],
        out_shape=[
            jax.ShapeDtypeStruct((8, 128), jnp.float32),
            jax.ShapeDtypeStruct((8, 128), jnp.float32),
        ],
    )(u, acca, accb, accn, wge, wgx, wgu, bg, weu)


def _edge_body(e, g, srcr, lo, wee, uterm, e_out, acc_out):
    i = pl.program_id(0)
    code_row = jnp.zeros((1, _BE), jnp.float32)
    for gidx in range(1, _B):
        code_row = code_row + (srcr[...] >= lo[0:1, gidx:gidx + 1]).astype(
            jnp.float32)
    oh = (_tcol(code_row) == _iota8f(_BE)).astype(jnp.float32)
    en = jnp.maximum(g[...] + e[...] @ wee[...] + oh @ uterm[...], 0.0)
    e_out[...] = en
    part = lax.dot_general(oh, en, (((0,), (0,)), ((), ())))
    cnt = lax.dot_general(oh, jnp.ones((_BE, 128), jnp.float32),
                          (((0,), (0,)), ((), ())))
    blk = jnp.concatenate([part, cnt], axis=0)

    @pl.when(i == 0)
    def _():
        acc_out[...] = blk

    @pl.when(i > 0)
    def _():
        acc_out[...] = acc_out[...] + blk


def _edge(e, g, src2f, lo, wee, uterm, co):
    ie = e.shape[1]
    return pl.pallas_call(
        _edge_body,
        grid=(_NBEC,),
        in_specs=[
            pl.BlockSpec((_BE, ie), lambda i: (i, 0)),
            pl.BlockSpec((_BE, 128), lambda i: (i, 0)),
            pl.BlockSpec((1, _BE), lambda i: (0, co + i)),
            _full_spec((1, 8)),
            _full_spec(wee.shape),
            _full_spec((8, 128)),
        ],
        out_specs=[
            pl.BlockSpec((_BE, 128), lambda i: (i, 0)),
            pl.BlockSpec((16, 128), lambda i: (0, 0)),
        ],
        out_shape=[
            jax.ShapeDtypeStruct((_EC, 128), jnp.float32),
            jax.ShapeDtypeStruct((16, 128), jnp.float32),
        ],
    )(e, g, src2f, lo, wee, uterm)


def _node_body(x, spa, spb, dp, batchr, u, wnx, wnm, wnu, bn, wr, br,
               x_out, acc_out, t_out, last):
    i = pl.program_id(0)
    deg = jnp.maximum(dp[0, :, 0:1] + dp[1, :, 0:1], 1.0)
    msg = (spa[0] + spa[1] + spb[0] + spb[1]) / deg
    uterm = u[0:8, 0:32] @ wnu[...]
    bcol = batchr[0].astype(jnp.float32)
    oh = (bcol == _iota8f(_BN)).astype(jnp.float32)
    xn = jnp.maximum(
        x[...] @ wnx[...] + msg @ wnm[...] + oh @ uterm + bn[...], 0.0)
    if last:
        x_out[...] = jax.nn.sigmoid(xn @ wr[...] + br[...])
    else:
        x_out[...] = xn
        t_out[...] = xn @ wr[...] + br[...]
    part = lax.dot_general(oh, xn, (((0,), (0,)), ((), ())))
    cnt = lax.dot_general(oh, jnp.ones((_BN, 128), jnp.float32),
                          (((0,), (0,)), ((), ())))
    blk = jnp.concatenate([part, cnt], axis=0)

    @pl.when(i == 0)
    def _():
        acc_out[...] = blk

    @pl.when(i > 0)
    def _():
        acc_out[...] = acc_out[...] + blk


def _node(x, spa, spb, dp, batch_r, u, wnx, wnm, wnu, bn, wr, br, last):
    return pl.pallas_call(
        functools.partial(_node_body, last=last),
        grid=(_NBN,),
        in_specs=[
            pl.BlockSpec((_BN, 128), lambda i: (i, 0)),
            pl.BlockSpec((2, _BN, 128), lambda i: (0, i, 0)),
            pl.BlockSpec((2, _BN, 128), lambda i: (0, i, 0)),
            pl.BlockSpec((2, _BN, 128), lambda i: (0, i, 0)),
            pl.BlockSpec((1, _BN, 1), lambda i: (i, 0, 0)),
            _full_spec((8, 128)),
            _full_spec(wnx.shape), _full_spec(wnm.shape),
            _full_spec(wnu.shape), _full_spec(bn.shape),
            _full_spec(wr.shape), _full_spec(br.shape),
        ],
        out_specs=[
            pl.BlockSpec((_BN, 128), lambda i: (i, 0)),
            pl.BlockSpec((16, 128), lambda i: (0, 0)),
            pl.BlockSpec((_BN, 128), lambda i: (i, 0)),
        ],
        out_shape=[
            jax.ShapeDtypeStruct((_N, 128), jnp.float32),
            jax.ShapeDtypeStruct((16, 128), jnp.float32),
            jax.ShapeDtypeStruct((_N, 128), jnp.float32),
        ],
    )(x, spa, spb, dp, batch_r, u, wnx, wnm, wnu, bn, wr, br)


def _final_body(u, acca, accb, accn, wge, wgx, wgu, bg, wu, bu, out):
    un8 = _glob_update(u, acca[...] + accb[...], accn[...],
                       wge, wgx, wgu, bg)
    y = jax.nn.sigmoid(un8[0:4, :] @ wu[...] + bu[...])
    out[...] = jnp.concatenate([y, jnp.zeros((4, 128), jnp.float32)], axis=0)


def _final(u, acca, accb, accn, wge, wgx, wgu, bg, wu, bu):
    return pl.pallas_call(
        _final_body,
        grid=(1,),
        in_specs=[
            _full_spec((8, 128)), _full_spec((16, 128)),
            _full_spec((16, 128)), _full_spec((16, 128)),
            _full_spec(wge.shape), _full_spec(wgx.shape),
            _full_spec(wgu.shape), _full_spec(bg.shape),
            _full_spec(wu.shape), _full_spec(bu.shape),
        ],
        out_specs=pl.BlockSpec((8, 128), lambda i: (0, 0)),
        out_shape=jax.ShapeDtypeStruct((8, 128), jnp.float32),
    )(u, acca, accb, accn, wge, wgx, wgu, bg, wu, bu)


_MESH = dict(core_axis_name="core", subcore_axis_name="subcore")


def _sc_gather(table, idx2, co):
    mesh = plsc.VectorSubcoreMesh(**_MESH)

    @functools.partial(
        pl.kernel,
        out_type=jax.ShapeDtypeStruct((_EC, 128), jnp.float32),
        mesh=mesh)
    def k(x_hbm, i_hbm, o_hbm):
        def body(i_vmem, o_vmem):
            pltpu.sync_copy(x_hbm.at[i_vmem.at[0]], o_vmem)

        pltpu.emit_pipeline(
            body,
            grid=(_EC // _GW,),
            in_specs=[pl.BlockSpec((1, _GW), lambda i: (0, co + i))],
            out_specs=[pl.BlockSpec((_GW, 128), lambda i: (i, 0))],
            core_axis_name=("core", "subcore"),
            dimension_semantics=(pltpu.PARALLEL,),
        )(i_hbm, o_hbm)

    return k(table, idx2)


def _sc_scatter_add(vals, idx2, zrows, co):
    mesh = plsc.VectorSubcoreMesh(**_MESH)

    @functools.partial(
        pl.kernel,
        out_type=jax.ShapeDtypeStruct((2, _N, 128), jnp.float32),
        mesh=mesh,
        scratch_types=[pltpu.VMEM_SHARED((_N, 128), jnp.float32)])
    def k(v_hbm, i_hbm, z_hbm, o_hbm, acc):
        c = lax.axis_index("core")
        s = lax.axis_index("subcore")

        @pl.loop(s, _NZCH, step=16)
        def _(i):
            pltpu.sync_copy(z_hbm, acc.at[pl.ds(i * _ZCH, _ZCH)])

        plsc.subcore_barrier()

        def body(v_vmem, i_vmem):
            pltpu.sync_copy(v_vmem, acc.at[i_vmem.at[0]], add=True)

        pltpu.emit_pipeline(
            body,
            grid=(_EC // _GW,),
            in_specs=[
                pl.BlockSpec((_GW, 128), lambda i: (i, 0)),
                pl.BlockSpec((1, _GW), lambda i: (0, co + i)),
            ],
            out_specs=[],
            core_axis_name=("core", "subcore"),
            dimension_semantics=(pltpu.PARALLEL,),
        )(v_hbm, i_hbm)
        plsc.subcore_barrier()

        @pl.loop(s, _NZCH, step=16)
        def _(i):
            pltpu.sync_copy(acc.at[pl.ds(i * _ZCH, _ZCH)],
                            o_hbm.at[c, pl.ds(i * _ZCH, _ZCH)])

    return k(vals, idx2, zrows)


def _sc_degree(idx2, zrows16):
    mesh = plsc.VectorSubcoreMesh(**_MESH)

    @functools.partial(
        pl.kernel,
        out_type=jax.ShapeDtypeStruct((2, _N, 128), jnp.float32),
        mesh=mesh,
        scratch_types=[pltpu.VMEM_SHARED((_N, 128), jnp.float32),
                       pltpu.VMEM((_GW, 128), jnp.float32)])
    def k(i_hbm, z_hbm, o_hbm, acc, ones_v):
        c = lax.axis_index("core")
        s = lax.axis_index("subcore")

        @pl.loop(0, _GW)
        def _(r):
            @pl.loop(0, 128, step=16)
            def _(j):
                ones_v[r, pl.ds(j, 16)] = jnp.ones((16,), jnp.float32)

        @pl.loop(s, _NZCH, step=16)
        def _(i):
            pltpu.sync_copy(z_hbm, acc.at[pl.ds(i * _ZCH, _ZCH)])

        plsc.subcore_barrier()

        def body(i_vmem):
            pltpu.sync_copy(ones_v, acc.at[i_vmem.at[0]], add=True)

        pltpu.emit_pipeline(
            body,
            grid=(_E // _GW,),
            in_specs=[pl.BlockSpec((1, _GW), lambda i: (0, i))],
            out_specs=[],
            core_axis_name=("core", "subcore"),
            dimension_semantics=(pltpu.PARALLEL,),
        )(i_hbm)
        plsc.subcore_barrier()

        @pl.loop(s, _NZCH, step=16)
        def _(i):
            pltpu.sync_copy(acc.at[pl.ds(i * _ZCH, _ZCH)],
                            o_hbm.at[c, pl.ds(i * _ZCH, _ZCH)])

    return k(idx2, zrows16)


def kernel(x, msa_feats, edge_attr, distances, edge_index, batch, params):
    p = params
    src = edge_index[0]
    dst = edge_index[1]

    xin = jnp.concatenate([x, msa_feats], axis=1)
    src2 = src.reshape(1, _E)
    dst2 = dst.reshape(1, _E)
    src2f = src.astype(jnp.float32).reshape(1, _E)
    dst2f = dst.astype(jnp.float32).reshape(1, _E)
    dist2 = distances.reshape(1, _E)
    eaT = edge_attr.T
    batch_r = batch.reshape(_NBN, _BN, 1)

    lo4 = jnp.searchsorted(batch, jnp.arange(_B, dtype=jnp.int32)).astype(
        jnp.float32)
    lo = jnp.concatenate([lo4, jnp.full((4,), float(_N + 1), jnp.float32)]
                         ).reshape(1, 8)

    zrows = jnp.zeros((_ZCH, 128), jnp.float32)

    def row(b):
        return b.reshape(1, -1)

    (enw1, enb1), (enw2, enb2) = p["enc_node"]
    (eew1, eeb1), (eew2, eeb2) = p["enc_edge"]

    we0, be0 = p["mp"][0][0]
    x_h, t = _enc_node(xin, enw1, row(enb1), enw2, row(enb2),
                       we0[0:128], row(be0))
    e_ch = [_enc_edge(eaT, dist2, src2f, dst2f,
                      eew1, row(eeb1), eew2, row(eeb2), p["sep_table"],
                      c * _NBEC)
            for c in range(_NC)]

    degp = _sc_degree(dst2, zrows)

    u = jnp.zeros((8, 128), jnp.float32)
    wr, br = p["ro_node"]
    wr_pad = jnp.pad(wr, ((0, 0), (0, 126)))
    br_pad = jnp.pad(row(br), ((0, 0), (0, 126)))

    uterm = jnp.zeros((8, 128), jnp.float32)
    acc_e = acc_n = None
    for li in range(6):
        (we, be), (wn, bn), (wg, bg) = p["mp"][li]
        ie = 144 if li == 0 else 128
        we_e = we[128:128 + ie]
        we_u = we[128 + ie:]

        if li > 0:
            wg_p, bg_p = p["mp"][li - 1][2]
            u, uterm = _uker(u, acc_e[0], acc_e[1], acc_n,
                             wg_p[0:128], wg_p[128:256], wg_p[256:288],
                             row(bg_p), we_u)

        g_ch = [_sc_gather(t, src2, c * (_EC // _GW)) for c in range(_NC)]
        acc_e, sp_ch = [], []
        for c in range(_NC):
            en, acc = _edge(e_ch[c], g_ch[c], src2f, lo, we_e, uterm,
                            c * _NBEC)
            e_ch[c] = en
            acc_e.append(acc)
            sp_ch.append(_sc_scatter_add(en, dst2, zrows, c * (_EC // _GW)))
        last = li == 5
        if last:
            wnext, bnext = wr_pad, br_pad
        else:
            wnext = p["mp"][li + 1][0][0][0:128]
            bnext = row(p["mp"][li + 1][0][1])
        x_h, acc_n, t = _node(
            x_h, sp_ch[0], sp_ch[1], degp, batch_r, u,
            wn[0:128], wn[128:256], wn[256:288], row(bn),
            wnext, bnext, last)

    (wg_l, bg_l) = p["mp"][5][2]
    wu, bu = p["ro_glob"]
    wu_pad = jnp.pad(wu, ((0, 0), (0, 123)))
    bu_pad = jnp.pad(row(bu), ((0, 0), (0, 123)))
    yg = _final(u, acc_e[0], acc_e[1], acc_n,
                wg_l[0:128], wg_l[128:256], wg_l[256:288],
                row(bg_l), wu_pad, bu_pad)

    return x_h[:, 0:2], yg[0:4, 0:5]
